# Initial kernel scaffold; baseline (speedup 1.0000x reference)
import jax
import jax.numpy as jnp
from jax.experimental import pallas as pl


def kernel(atom_list, param_list, mass_list, process_num):
    raise NotImplementedError("write your pallas kernel here")


# speedup vs baseline: 1.2676x; 1.2676x over previous
import jax
import jax.numpy as jnp
from jax.experimental import pallas as pl

_N = 2048
_CUT = 6.0
_BI = 64


def _pair_f(s, C, k, c, inv_re):
    e = jnp.exp(-k * (s - 1.0))
    t = s - c
    t2 = t * t
    t4 = t2 * t2
    t8 = t4 * t4
    t16 = t8 * t8
    t20 = t16 * t4
    t19 = t16 * t2 * t
    invD = 1.0 / (1.0 + t20)
    f = C * e * invD
    dfdr = f * (-k - 20.0 * t19 * invD) * inv_re
    return f, dfdr


def _body(atom_ref, atomT_ref, par_ref, parT_ref, mass_ref,
          frho_ref, pe_ref, ax_ref, ay_ref, az_ref):
    i = pl.program_id(0)

    xi = atom_ref[:, 0:1]
    yi = atom_ref[:, 1:2]
    zi = atom_ref[:, 2:3]
    xj = atomT_ref[0:1, :]
    yj = atomT_ref[1:2, :]
    zj = atomT_ref[2:3, :]
    dx = xj - xi
    dy = yj - yi
    dz = zj - zi
    r = jnp.sqrt(dx * dx + dy * dy + dz * dz + 1e-12)

    rows = i * _BI + jax.lax.broadcasted_iota(jnp.int32, (_BI, _N), 0)
    cols = jax.lax.broadcasted_iota(jnp.int32, (_BI, _N), 1)
    mask = (r <= _CUT) & (rows != cols)
    rs = jnp.where(mask, r, 1.0)
    inv_rs = 1.0 / rs

    re_i = par_ref[:, 0:1]
    fe_i = par_ref[:, 1:2]
    al_i = par_ref[:, 4:5]
    be_i = par_ref[:, 5:6]
    a_i = par_ref[:, 6:7]
    b_i = par_ref[:, 7:8]
    ka_i = par_ref[:, 8:9]
    la_i = par_ref[:, 9:10]
    re_j = parT_ref[0:1, :]
    fe_j = parT_ref[1:2, :]
    al_j = parT_ref[4:5, :]
    be_j = parT_ref[5:6, :]
    a_j = parT_ref[6:7, :]
    b_j = parT_ref[7:8, :]
    ka_j = parT_ref[8:9, :]
    la_j = parT_ref[9:10, :]

    inv_re_i = 1.0 / re_i
    inv_re_j = 1.0 / re_j
    si = rs * inv_re_i
    sj = rs * inv_re_j

    fr0, dfr0 = _pair_f(si, fe_i, be_i, la_i, inv_re_i)
    l0, dl0 = _pair_f(si, a_i, al_i, ka_i, inv_re_i)
    fr1, dfr1 = _pair_f(sj, fe_j, be_j, la_j, inv_re_j)
    l1, dl1 = _pair_f(sj, a_j, al_j, ka_j, inv_re_j)

    c0 = b_i / fe_i
    phi0 = l0 - c0 * fr0
    dphi0 = dl0 - c0 * dfr0
    c1 = b_j / fe_j
    phi1 = l1 - c1 * fr1
    dphi1 = dl1 - c1 * dfr1

    inv_fr0 = 1.0 / fr0
    inv_fr1 = 1.0 / fr1
    R = fr1 * inv_fr0
    Q = fr0 * inv_fr1
    phi01 = 0.5 * (R * phi0 + Q * phi1)
    dR = (dfr1 * fr0 - fr1 * dfr0) * (inv_fr0 * inv_fr0)
    dQ = (dfr0 * fr1 - fr0 * dfr1) * (inv_fr1 * inv_fr1)
    dphi01 = 0.5 * (dR * phi0 + R * dphi0 + dQ * phi1 + Q * dphi1)

    zero = jnp.zeros_like(r)
    rho = jnp.sum(jnp.where(mask, fr1, zero), axis=1, keepdims=True)
    pe = jnp.sum(jnp.where(mask, phi01, zero), axis=1, keepdims=True)
    w1 = jnp.where(mask, dfr1 * inv_rs, zero)
    w2 = jnp.where(mask, dphi01 * inv_rs, zero)
    s1x = jnp.sum(w1 * dx, axis=1, keepdims=True)
    s1y = jnp.sum(w1 * dy, axis=1, keepdims=True)
    s1z = jnp.sum(w1 * dz, axis=1, keepdims=True)
    s2x = jnp.sum(w2 * dx, axis=1, keepdims=True)
    s2y = jnp.sum(w2 * dy, axis=1, keepdims=True)
    s2z = jnp.sum(w2 * dz, axis=1, keepdims=True)

    f_n0 = par_ref[:, 10:11]
    f_n1 = par_ref[:, 11:12]
    f_n2 = par_ref[:, 12:13]
    f_n3 = par_ref[:, 13:14]
    f_0 = par_ref[:, 14:15]
    f_1 = par_ref[:, 15:16]
    f_2 = par_ref[:, 16:17]
    f_3 = par_ref[:, 17:18]
    fe_e = par_ref[:, 19:20]
    rho_n = par_ref[:, 20:21]
    rho_e = par_ref[:, 2:3]
    rho_0 = par_ref[:, 21:22]
    rho_s = par_ref[:, 3:4]
    eta = par_ref[:, 18:19]

    rho_c = jnp.maximum(rho, 1e-8)
    inv_rho_n = 1.0 / rho_n
    t = rho_c * inv_rho_n - 1.0
    b1 = f_n0 + t * (f_n1 + t * (f_n2 + t * f_n3))
    db1 = (f_n1 + t * (2.0 * f_n2 + t * (3.0 * f_n3))) * inv_rho_n
    inv_rho_e = 1.0 / rho_e
    u = rho_c * inv_rho_e - 1.0
    b2 = f_0 + u * (f_1 + u * (f_2 + u * f_3))
    db2 = (f_1 + u * (2.0 * f_2 + u * (3.0 * f_3))) * inv_rho_e
    x = rho_c / rho_s
    lnx = jnp.log(x)
    xeta = jnp.exp(eta * lnx)
    b3 = fe_e * (1.0 - eta * lnx) * xeta
    db3 = -fe_e * eta * eta * lnx * xeta / rho_c
    F = jnp.where(rho_c < rho_n, b1, jnp.where(rho_c < rho_0, b2, b3))
    dF = jnp.where(rho_c < rho_n, db1, jnp.where(rho_c < rho_0, db2, db3))
    dF = jnp.where(rho > 1e-8, dF, 0.0)

    inv_m = 1.0 / mass_ref[:, 0:1]
    frho_ref[:, :] = F
    pe_ref[:, :] = pe
    ax_ref[:, :] = (dF * s1x + 0.25 * s2x) * inv_m
    ay_ref[:, :] = (dF * s1y + 0.25 * s2y) * inv_m
    az_ref[:, :] = (dF * s1z + 0.25 * s2z) * inv_m


def kernel(atom_list, param_list, mass_list, process_num):
    del process_num
    atomT = atom_list.T
    parT = param_list.T
    mass2 = mass_list[:, None]
    grid = (_N // _BI,)
    out = pl.pallas_call(
        _body,
        grid=grid,
        in_specs=[
            pl.BlockSpec((_BI, 3), lambda i: (i, 0)),
            pl.BlockSpec((3, _N), lambda i: (0, 0)),
            pl.BlockSpec((_BI, 22), lambda i: (i, 0)),
            pl.BlockSpec((22, _N), lambda i: (0, 0)),
            pl.BlockSpec((_BI, 1), lambda i: (i, 0)),
        ],
        out_specs=[
            pl.BlockSpec((_BI, 1), lambda i: (i, 0)),
            pl.BlockSpec((_BI, 1), lambda i: (i, 0)),
            pl.BlockSpec((_BI, 1), lambda i: (i, 0)),
            pl.BlockSpec((_BI, 1), lambda i: (i, 0)),
            pl.BlockSpec((_BI, 1), lambda i: (i, 0)),
        ],
        out_shape=[jax.ShapeDtypeStruct((_N, 1), jnp.float32)] * 5,
    )(atom_list, atomT, param_list, parT, mass2)
    frho, pe, ax, ay, az = out
    frho = frho[:, 0]
    pe = pe[:, 0]
    acc = jnp.concatenate([out[2], out[3], out[4]], axis=1)
    e_total = jnp.sum(frho) + jnp.sum(pe)
    return e_total, frho, pe, acc


# speedup vs baseline: 1.2900x; 1.0177x over previous
import jax
import jax.numpy as jnp
from jax.experimental import pallas as pl

_N = 2048
_CUT = 6.0
_BI = 64


def _pair_f(s, C, k, c, inv_re):
    e = jnp.exp(-k * (s - 1.0))
    t = s - c
    t2 = t * t
    t4 = t2 * t2
    t8 = t4 * t4
    t16 = t8 * t8
    t20 = t16 * t4
    t19 = t16 * t2 * t
    invD = 1.0 / (1.0 + t20)
    f = C * e * invD
    g = (-k - 20.0 * t19 * invD) * inv_re
    return f, g


def _body(atom_ref, atomT_ref, par_ref, parT_ref, mass_ref, rid_ref, cid_ref,
          frho_ref, pe_ref, ax_ref, ay_ref, az_ref):
    xi = atom_ref[:, 0:1]
    yi = atom_ref[:, 1:2]
    zi = atom_ref[:, 2:3]
    xj = atomT_ref[0:1, :]
    yj = atomT_ref[1:2, :]
    zj = atomT_ref[2:3, :]
    dx = xj - xi
    dy = yj - yi
    dz = zj - zi
    r = jnp.sqrt(dx * dx + dy * dy + dz * dz + 1e-12)

    mask = (r <= _CUT) & (rid_ref[:, 0:1] != cid_ref[0:1, :])
    rs = jnp.where(mask, r, 1.0)
    inv_rs = 1.0 / rs

    re_i = par_ref[:, 0:1]
    fe_i = par_ref[:, 1:2]
    al_i = par_ref[:, 4:5]
    be_i = par_ref[:, 5:6]
    a_i = par_ref[:, 6:7]
    b_i = par_ref[:, 7:8]
    ka_i = par_ref[:, 8:9]
    la_i = par_ref[:, 9:10]
    re_j = parT_ref[0:1, :]
    fe_j = parT_ref[1:2, :]
    al_j = parT_ref[4:5, :]
    be_j = parT_ref[5:6, :]
    a_j = parT_ref[6:7, :]
    b_j = parT_ref[7:8, :]
    ka_j = parT_ref[8:9, :]
    la_j = parT_ref[9:10, :]

    inv_re_i = 1.0 / re_i
    inv_re_j = 1.0 / re_j
    si = rs * inv_re_i
    sj = rs * inv_re_j

    fr0, g0 = _pair_f(si, fe_i, be_i, la_i, inv_re_i)
    l0, gl0 = _pair_f(si, a_i, al_i, ka_i, inv_re_i)
    fr1, g1 = _pair_f(sj, fe_j, be_j, la_j, inv_re_j)
    l1, gl1 = _pair_f(sj, a_j, al_j, ka_j, inv_re_j)
    dfr1 = fr1 * g1

    c0 = b_i / fe_i
    c0fr0 = c0 * fr0
    phi0 = l0 - c0fr0
    dphi0 = l0 * gl0 - c0fr0 * g0
    c1 = b_j / fe_j
    c1fr1 = c1 * fr1
    phi1 = l1 - c1fr1
    dphi1 = l1 * gl1 - c1fr1 * g1

    inv_fr0 = 1.0 / fr0
    inv_fr1 = 1.0 / fr1
    R = fr1 * inv_fr0
    Q = fr0 * inv_fr1
    phi01 = 0.5 * (R * phi0 + Q * phi1)
    gd = g1 - g0
    dR = R * gd
    dQ = -(Q * gd)
    dphi01 = 0.5 * (dR * phi0 + R * dphi0 + dQ * phi1 + Q * dphi1)

    zero = jnp.zeros_like(r)
    rho = jnp.sum(jnp.where(mask, fr1, zero), axis=1, keepdims=True)
    pe = jnp.sum(jnp.where(mask, phi01, zero), axis=1, keepdims=True)
    w1 = jnp.where(mask, dfr1 * inv_rs, zero)
    w2 = jnp.where(mask, dphi01 * inv_rs, zero)
    s1x = jnp.sum(w1 * dx, axis=1, keepdims=True)
    s1y = jnp.sum(w1 * dy, axis=1, keepdims=True)
    s1z = jnp.sum(w1 * dz, axis=1, keepdims=True)
    s2x = jnp.sum(w2 * dx, axis=1, keepdims=True)
    s2y = jnp.sum(w2 * dy, axis=1, keepdims=True)
    s2z = jnp.sum(w2 * dz, axis=1, keepdims=True)

    f_n0 = par_ref[:, 10:11]
    f_n1 = par_ref[:, 11:12]
    f_n2 = par_ref[:, 12:13]
    f_n3 = par_ref[:, 13:14]
    f_0 = par_ref[:, 14:15]
    f_1 = par_ref[:, 15:16]
    f_2 = par_ref[:, 16:17]
    f_3 = par_ref[:, 17:18]
    fe_e = par_ref[:, 19:20]
    rho_n = par_ref[:, 20:21]
    rho_e = par_ref[:, 2:3]
    rho_0 = par_ref[:, 21:22]
    rho_s = par_ref[:, 3:4]
    eta = par_ref[:, 18:19]

    rho_c = jnp.maximum(rho, 1e-8)
    inv_rho_n = 1.0 / rho_n
    t = rho_c * inv_rho_n - 1.0
    b1 = f_n0 + t * (f_n1 + t * (f_n2 + t * f_n3))
    db1 = (f_n1 + t * (2.0 * f_n2 + t * (3.0 * f_n3))) * inv_rho_n
    inv_rho_e = 1.0 / rho_e
    u = rho_c * inv_rho_e - 1.0
    b2 = f_0 + u * (f_1 + u * (f_2 + u * f_3))
    db2 = (f_1 + u * (2.0 * f_2 + u * (3.0 * f_3))) * inv_rho_e
    x = rho_c / rho_s
    lnx = jnp.log(x)
    xeta = jnp.exp(eta * lnx)
    b3 = fe_e * (1.0 - eta * lnx) * xeta
    db3 = -fe_e * eta * eta * lnx * xeta / rho_c
    F = jnp.where(rho_c < rho_n, b1, jnp.where(rho_c < rho_0, b2, b3))
    dF = jnp.where(rho_c < rho_n, db1, jnp.where(rho_c < rho_0, db2, db3))
    dF = jnp.where(rho > 1e-8, dF, 0.0)

    inv_m = 1.0 / mass_ref[:, 0:1]
    frho_ref[:, :] = F
    pe_ref[:, :] = pe
    ax_ref[:, :] = (dF * s1x + 0.25 * s2x) * inv_m
    ay_ref[:, :] = (dF * s1y + 0.25 * s2y) * inv_m
    az_ref[:, :] = (dF * s1z + 0.25 * s2z) * inv_m


def kernel(atom_list, param_list, mass_list, process_num):
    del process_num
    atomT = atom_list.T
    parT = param_list.T
    mass2 = mass_list[:, None]
    ids = jnp.arange(_N, dtype=jnp.float32)
    rid = ids[:, None]
    cid = ids[None, :]
    grid = (_N // _BI,)
    out = pl.pallas_call(
        _body,
        grid=grid,
        in_specs=[
            pl.BlockSpec((_BI, 3), lambda i: (i, 0)),
            pl.BlockSpec((3, _N), lambda i: (0, 0)),
            pl.BlockSpec((_BI, 22), lambda i: (i, 0)),
            pl.BlockSpec((22, _N), lambda i: (0, 0)),
            pl.BlockSpec((_BI, 1), lambda i: (i, 0)),
            pl.BlockSpec((_BI, 1), lambda i: (i, 0)),
            pl.BlockSpec((1, _N), lambda i: (0, 0)),
        ],
        out_specs=[
            pl.BlockSpec((_BI, 1), lambda i: (i, 0)),
            pl.BlockSpec((_BI, 1), lambda i: (i, 0)),
            pl.BlockSpec((_BI, 1), lambda i: (i, 0)),
            pl.BlockSpec((_BI, 1), lambda i: (i, 0)),
            pl.BlockSpec((_BI, 1), lambda i: (i, 0)),
        ],
        out_shape=[jax.ShapeDtypeStruct((_N, 1), jnp.float32)] * 5,
    )(atom_list, atomT, param_list, parT, mass2, rid, cid)
    frho, pe, ax, ay, az = out
    frho = frho[:, 0]
    pe = pe[:, 0]
    acc = jnp.concatenate([out[2], out[3], out[4]], axis=1)
    e_total = jnp.sum(frho) + jnp.sum(pe)
    return e_total, frho, pe, acc


# speedup vs baseline: 1.3061x; 1.0125x over previous
import jax
import jax.numpy as jnp
from jax.experimental import pallas as pl

_N = 2048
_CUT = 6.0
_BI = 128


def _pair_f(s, C, k, c, inv_re):
    e = jnp.exp(-k * (s - 1.0))
    t = s - c
    t2 = t * t
    t4 = t2 * t2
    t8 = t4 * t4
    t16 = t8 * t8
    t20 = t16 * t4
    t19 = t16 * t2 * t
    invD = 1.0 / (1.0 + t20)
    f = C * e * invD
    g = (-k - 20.0 * t19 * invD) * inv_re
    return f, g


def _body(atom_ref, atomT_ref, par_ref, parT_ref, mass_ref, rid_ref, cid_ref,
          frho_ref, pe_ref, ax_ref, ay_ref, az_ref):
    xi = atom_ref[:, 0:1]
    yi = atom_ref[:, 1:2]
    zi = atom_ref[:, 2:3]
    xj = atomT_ref[0:1, :]
    yj = atomT_ref[1:2, :]
    zj = atomT_ref[2:3, :]
    dx = xj - xi
    dy = yj - yi
    dz = zj - zi
    r = jnp.sqrt(dx * dx + dy * dy + dz * dz + 1e-12)

    mask = (r <= _CUT) & (rid_ref[:, 0:1] != cid_ref[0:1, :])
    rs = jnp.where(mask, r, 1.0)
    inv_rs = 1.0 / rs

    re_i = par_ref[:, 0:1]
    fe_i = par_ref[:, 1:2]
    al_i = par_ref[:, 4:5]
    be_i = par_ref[:, 5:6]
    a_i = par_ref[:, 6:7]
    b_i = par_ref[:, 7:8]
    ka_i = par_ref[:, 8:9]
    la_i = par_ref[:, 9:10]
    re_j = parT_ref[0:1, :]
    fe_j = parT_ref[1:2, :]
    al_j = parT_ref[4:5, :]
    be_j = parT_ref[5:6, :]
    a_j = parT_ref[6:7, :]
    b_j = parT_ref[7:8, :]
    ka_j = parT_ref[8:9, :]
    la_j = parT_ref[9:10, :]

    inv_re_i = 1.0 / re_i
    inv_re_j = 1.0 / re_j
    si = rs * inv_re_i
    sj = rs * inv_re_j

    fr0, g0 = _pair_f(si, fe_i, be_i, la_i, inv_re_i)
    l0, gl0 = _pair_f(si, a_i, al_i, ka_i, inv_re_i)
    fr1, g1 = _pair_f(sj, fe_j, be_j, la_j, inv_re_j)
    l1, gl1 = _pair_f(sj, a_j, al_j, ka_j, inv_re_j)
    dfr1 = fr1 * g1

    c0 = b_i / fe_i
    c0fr0 = c0 * fr0
    phi0 = l0 - c0fr0
    dphi0 = l0 * gl0 - c0fr0 * g0
    c1 = b_j / fe_j
    c1fr1 = c1 * fr1
    phi1 = l1 - c1fr1
    dphi1 = l1 * gl1 - c1fr1 * g1

    inv_fr0 = 1.0 / fr0
    inv_fr1 = 1.0 / fr1
    R = fr1 * inv_fr0
    Q = fr0 * inv_fr1
    phi01 = 0.5 * (R * phi0 + Q * phi1)
    gd = g1 - g0
    dR = R * gd
    dQ = -(Q * gd)
    dphi01 = 0.5 * (dR * phi0 + R * dphi0 + dQ * phi1 + Q * dphi1)

    zero = jnp.zeros_like(r)
    rho = jnp.sum(jnp.where(mask, fr1, zero), axis=1, keepdims=True)
    pe = jnp.sum(jnp.where(mask, phi01, zero), axis=1, keepdims=True)
    w1 = jnp.where(mask, dfr1 * inv_rs, zero)
    w2 = jnp.where(mask, dphi01 * inv_rs, zero)
    s1x = jnp.sum(w1 * dx, axis=1, keepdims=True)
    s1y = jnp.sum(w1 * dy, axis=1, keepdims=True)
    s1z = jnp.sum(w1 * dz, axis=1, keepdims=True)
    s2x = jnp.sum(w2 * dx, axis=1, keepdims=True)
    s2y = jnp.sum(w2 * dy, axis=1, keepdims=True)
    s2z = jnp.sum(w2 * dz, axis=1, keepdims=True)

    f_n0 = par_ref[:, 10:11]
    f_n1 = par_ref[:, 11:12]
    f_n2 = par_ref[:, 12:13]
    f_n3 = par_ref[:, 13:14]
    f_0 = par_ref[:, 14:15]
    f_1 = par_ref[:, 15:16]
    f_2 = par_ref[:, 16:17]
    f_3 = par_ref[:, 17:18]
    fe_e = par_ref[:, 19:20]
    rho_n = par_ref[:, 20:21]
    rho_e = par_ref[:, 2:3]
    rho_0 = par_ref[:, 21:22]
    rho_s = par_ref[:, 3:4]
    eta = par_ref[:, 18:19]

    rho_c = jnp.maximum(rho, 1e-8)
    inv_rho_n = 1.0 / rho_n
    t = rho_c * inv_rho_n - 1.0
    b1 = f_n0 + t * (f_n1 + t * (f_n2 + t * f_n3))
    db1 = (f_n1 + t * (2.0 * f_n2 + t * (3.0 * f_n3))) * inv_rho_n
    inv_rho_e = 1.0 / rho_e
    u = rho_c * inv_rho_e - 1.0
    b2 = f_0 + u * (f_1 + u * (f_2 + u * f_3))
    db2 = (f_1 + u * (2.0 * f_2 + u * (3.0 * f_3))) * inv_rho_e
    x = rho_c / rho_s
    lnx = jnp.log(x)
    xeta = jnp.exp(eta * lnx)
    b3 = fe_e * (1.0 - eta * lnx) * xeta
    db3 = -fe_e * eta * eta * lnx * xeta / rho_c
    F = jnp.where(rho_c < rho_n, b1, jnp.where(rho_c < rho_0, b2, b3))
    dF = jnp.where(rho_c < rho_n, db1, jnp.where(rho_c < rho_0, db2, db3))
    dF = jnp.where(rho > 1e-8, dF, 0.0)

    inv_m = 1.0 / mass_ref[:, 0:1]
    frho_ref[:, :] = F
    pe_ref[:, :] = pe
    ax_ref[:, :] = (dF * s1x + 0.25 * s2x) * inv_m
    ay_ref[:, :] = (dF * s1y + 0.25 * s2y) * inv_m
    az_ref[:, :] = (dF * s1z + 0.25 * s2z) * inv_m


def kernel(atom_list, param_list, mass_list, process_num):
    del process_num
    atomT = atom_list.T
    parT = param_list.T
    mass2 = mass_list[:, None]
    ids = jnp.arange(_N, dtype=jnp.float32)
    rid = ids[:, None]
    cid = ids[None, :]
    grid = (_N // _BI,)
    out = pl.pallas_call(
        _body,
        grid=grid,
        in_specs=[
            pl.BlockSpec((_BI, 3), lambda i: (i, 0)),
            pl.BlockSpec((3, _N), lambda i: (0, 0)),
            pl.BlockSpec((_BI, 22), lambda i: (i, 0)),
            pl.BlockSpec((22, _N), lambda i: (0, 0)),
            pl.BlockSpec((_BI, 1), lambda i: (i, 0)),
            pl.BlockSpec((_BI, 1), lambda i: (i, 0)),
            pl.BlockSpec((1, _N), lambda i: (0, 0)),
        ],
        out_specs=[
            pl.BlockSpec((_BI, 1), lambda i: (i, 0)),
            pl.BlockSpec((_BI, 1), lambda i: (i, 0)),
            pl.BlockSpec((_BI, 1), lambda i: (i, 0)),
            pl.BlockSpec((_BI, 1), lambda i: (i, 0)),
            pl.BlockSpec((_BI, 1), lambda i: (i, 0)),
        ],
        out_shape=[jax.ShapeDtypeStruct((_N, 1), jnp.float32)] * 5,
    )(atom_list, atomT, param_list, parT, mass2, rid, cid)
    frho, pe, ax, ay, az = out
    frho = frho[:, 0]
    pe = pe[:, 0]
    acc = jnp.concatenate([out[2], out[3], out[4]], axis=1)
    e_total = jnp.sum(frho) + jnp.sum(pe)
    return e_total, frho, pe, acc


# speedup vs baseline: 1.4206x; 1.0876x over previous
import jax
import jax.numpy as jnp
from jax.experimental import pallas as pl

_N = 2048
_CUT = 6.0
_BI = 256


def _pair_f(s, C, k, c, inv_re):
    e = jnp.exp(-k * (s - 1.0))
    t = s - c
    t2 = t * t
    t4 = t2 * t2
    t8 = t4 * t4
    t16 = t8 * t8
    t20 = t16 * t4
    t19 = t16 * t2 * t
    invD = 1.0 / (1.0 + t20)
    f = C * e * invD
    g = (-k - 20.0 * t19 * invD) * inv_re
    return f, g


def _body(atom_ref, atomT_ref, par_ref, parT_ref, mass_ref, rid_ref, cid_ref,
          frho_ref, pe_ref, ax_ref, ay_ref, az_ref):
    xi = atom_ref[:, 0:1]
    yi = atom_ref[:, 1:2]
    zi = atom_ref[:, 2:3]
    xj = atomT_ref[0:1, :]
    yj = atomT_ref[1:2, :]
    zj = atomT_ref[2:3, :]
    dx = xj - xi
    dy = yj - yi
    dz = zj - zi
    r = jnp.sqrt(dx * dx + dy * dy + dz * dz + 1e-12)

    mask = (r <= _CUT) & (rid_ref[:, 0:1] != cid_ref[0:1, :])
    rs = jnp.where(mask, r, 1.0)
    inv_rs = 1.0 / rs

    re_i = par_ref[:, 0:1]
    fe_i = par_ref[:, 1:2]
    al_i = par_ref[:, 4:5]
    be_i = par_ref[:, 5:6]
    a_i = par_ref[:, 6:7]
    b_i = par_ref[:, 7:8]
    ka_i = par_ref[:, 8:9]
    la_i = par_ref[:, 9:10]
    re_j = parT_ref[0:1, :]
    fe_j = parT_ref[1:2, :]
    al_j = parT_ref[4:5, :]
    be_j = parT_ref[5:6, :]
    a_j = parT_ref[6:7, :]
    b_j = parT_ref[7:8, :]
    ka_j = parT_ref[8:9, :]
    la_j = parT_ref[9:10, :]

    inv_re_i = 1.0 / re_i
    inv_re_j = 1.0 / re_j
    si = rs * inv_re_i
    sj = rs * inv_re_j

    fr0, g0 = _pair_f(si, fe_i, be_i, la_i, inv_re_i)
    l0, gl0 = _pair_f(si, a_i, al_i, ka_i, inv_re_i)
    fr1, g1 = _pair_f(sj, fe_j, be_j, la_j, inv_re_j)
    l1, gl1 = _pair_f(sj, a_j, al_j, ka_j, inv_re_j)
    dfr1 = fr1 * g1

    c0 = b_i / fe_i
    c0fr0 = c0 * fr0
    phi0 = l0 - c0fr0
    dphi0 = l0 * gl0 - c0fr0 * g0
    c1 = b_j / fe_j
    c1fr1 = c1 * fr1
    phi1 = l1 - c1fr1
    dphi1 = l1 * gl1 - c1fr1 * g1

    inv_fr0 = 1.0 / fr0
    inv_fr1 = 1.0 / fr1
    R = fr1 * inv_fr0
    Q = fr0 * inv_fr1
    phi01 = 0.5 * (R * phi0 + Q * phi1)
    gd = g1 - g0
    dR = R * gd
    dQ = -(Q * gd)
    dphi01 = 0.5 * (dR * phi0 + R * dphi0 + dQ * phi1 + Q * dphi1)

    zero = jnp.zeros_like(r)
    rho = jnp.sum(jnp.where(mask, fr1, zero), axis=1, keepdims=True)
    pe = jnp.sum(jnp.where(mask, phi01, zero), axis=1, keepdims=True)
    w1 = jnp.where(mask, dfr1 * inv_rs, zero)
    w2 = jnp.where(mask, dphi01 * inv_rs, zero)
    s1x = jnp.sum(w1 * dx, axis=1, keepdims=True)
    s1y = jnp.sum(w1 * dy, axis=1, keepdims=True)
    s1z = jnp.sum(w1 * dz, axis=1, keepdims=True)
    s2x = jnp.sum(w2 * dx, axis=1, keepdims=True)
    s2y = jnp.sum(w2 * dy, axis=1, keepdims=True)
    s2z = jnp.sum(w2 * dz, axis=1, keepdims=True)

    f_n0 = par_ref[:, 10:11]
    f_n1 = par_ref[:, 11:12]
    f_n2 = par_ref[:, 12:13]
    f_n3 = par_ref[:, 13:14]
    f_0 = par_ref[:, 14:15]
    f_1 = par_ref[:, 15:16]
    f_2 = par_ref[:, 16:17]
    f_3 = par_ref[:, 17:18]
    fe_e = par_ref[:, 19:20]
    rho_n = par_ref[:, 20:21]
    rho_e = par_ref[:, 2:3]
    rho_0 = par_ref[:, 21:22]
    rho_s = par_ref[:, 3:4]
    eta = par_ref[:, 18:19]

    rho_c = jnp.maximum(rho, 1e-8)
    inv_rho_n = 1.0 / rho_n
    t = rho_c * inv_rho_n - 1.0
    b1 = f_n0 + t * (f_n1 + t * (f_n2 + t * f_n3))
    db1 = (f_n1 + t * (2.0 * f_n2 + t * (3.0 * f_n3))) * inv_rho_n
    inv_rho_e = 1.0 / rho_e
    u = rho_c * inv_rho_e - 1.0
    b2 = f_0 + u * (f_1 + u * (f_2 + u * f_3))
    db2 = (f_1 + u * (2.0 * f_2 + u * (3.0 * f_3))) * inv_rho_e
    x = rho_c / rho_s
    lnx = jnp.log(x)
    xeta = jnp.exp(eta * lnx)
    b3 = fe_e * (1.0 - eta * lnx) * xeta
    db3 = -fe_e * eta * eta * lnx * xeta / rho_c
    F = jnp.where(rho_c < rho_n, b1, jnp.where(rho_c < rho_0, b2, b3))
    dF = jnp.where(rho_c < rho_n, db1, jnp.where(rho_c < rho_0, db2, db3))
    dF = jnp.where(rho > 1e-8, dF, 0.0)

    inv_m = 1.0 / mass_ref[:, 0:1]
    frho_ref[:, :] = F
    pe_ref[:, :] = pe
    ax_ref[:, :] = (dF * s1x + 0.25 * s2x) * inv_m
    ay_ref[:, :] = (dF * s1y + 0.25 * s2y) * inv_m
    az_ref[:, :] = (dF * s1z + 0.25 * s2z) * inv_m


def kernel(atom_list, param_list, mass_list, process_num):
    del process_num
    atomT = atom_list.T
    parT = param_list.T
    mass2 = mass_list[:, None]
    ids = jnp.arange(_N, dtype=jnp.float32)
    rid = ids[:, None]
    cid = ids[None, :]
    grid = (_N // _BI,)
    out = pl.pallas_call(
        _body,
        grid=grid,
        in_specs=[
            pl.BlockSpec((_BI, 3), lambda i: (i, 0)),
            pl.BlockSpec((3, _N), lambda i: (0, 0)),
            pl.BlockSpec((_BI, 22), lambda i: (i, 0)),
            pl.BlockSpec((22, _N), lambda i: (0, 0)),
            pl.BlockSpec((_BI, 1), lambda i: (i, 0)),
            pl.BlockSpec((_BI, 1), lambda i: (i, 0)),
            pl.BlockSpec((1, _N), lambda i: (0, 0)),
        ],
        out_specs=[
            pl.BlockSpec((_BI, 1), lambda i: (i, 0)),
            pl.BlockSpec((_BI, 1), lambda i: (i, 0)),
            pl.BlockSpec((_BI, 1), lambda i: (i, 0)),
            pl.BlockSpec((_BI, 1), lambda i: (i, 0)),
            pl.BlockSpec((_BI, 1), lambda i: (i, 0)),
        ],
        out_shape=[jax.ShapeDtypeStruct((_N, 1), jnp.float32)] * 5,
    )(atom_list, atomT, param_list, parT, mass2, rid, cid)
    frho, pe, ax, ay, az = out
    frho = frho[:, 0]
    pe = pe[:, 0]
    acc = jnp.concatenate([out[2], out[3], out[4]], axis=1)
    e_total = jnp.sum(frho) + jnp.sum(pe)
    return e_total, frho, pe, acc
